# Initial kernel scaffold; baseline (speedup 1.0000x reference)
import jax
import jax.numpy as jnp
from jax.experimental import pallas as pl


def kernel(x_user, x_content, edge_index, lstm_Wih, lstm_Whh, lstm_bih, lstm_bhh, Wc, bc, Wl0, Wr0, b0, Wl1, Wr1, b1, cls_W1, cls_b1, cls_W2, cls_b2):
    raise NotImplementedError("write your pallas kernel here")


# speedup vs baseline: 1.1363x; 1.1363x over previous
import functools

import jax
import jax.numpy as jnp
from jax import lax
from jax.experimental import pallas as pl
from jax.experimental.pallas import tpu as pltpu

N_USER = 10000
N_CONTENT = 10000
SEQ = 20
E_DIM = 128
H = 128
N_EDGES = 320000

BU = 1000
BD = 1000


def _dotT(a, b):
    return lax.dot_general(a, b, (((1,), (1,)), ((), ())),
                           preferred_element_type=jnp.float32)


def _lstm_body(x_ref, wih_ref, whh_ref, b_ref, u_ref):
    nb = x_ref.shape[0]
    h = jnp.zeros((nb, H), jnp.float32)
    c = jnp.zeros((nb, H), jnp.float32)
    for t in range(SEQ):
        z = _dotT(x_ref[:, t, :], wih_ref[...]) + _dotT(h, whh_ref[...]) + b_ref[...]
        i = jax.nn.sigmoid(z[:, 0:H])
        f = jax.nn.sigmoid(z[:, H:2 * H])
        g = jnp.tanh(z[:, 2 * H:3 * H])
        o = jax.nn.sigmoid(z[:, 3 * H:4 * H])
        c = f * c + i * g
        h = o * jnp.tanh(c)
    u_ref[...] = h


def _lstm_last(x, wih, whh, bias):
    grid = (N_USER // BU,)
    return pl.pallas_call(
        _lstm_body,
        grid=grid,
        in_specs=[
            pl.BlockSpec((BU, SEQ, E_DIM), lambda i: (i, 0, 0)),
            pl.BlockSpec((4 * H, E_DIM), lambda i: (0, 0)),
            pl.BlockSpec((4 * H, H), lambda i: (0, 0)),
            pl.BlockSpec((1, 4 * H), lambda i: (0, 0)),
        ],
        out_specs=pl.BlockSpec((BU, H), lambda i: (i, 0)),
        out_shape=jax.ShapeDtypeStruct((N_USER, H), jnp.float32),
    )(x, wih, whh, bias)


def _sage_dense_body(sum_ref, deg_ref, xdst_ref, wl_ref, wr_ref, b_ref, out_ref):
    mean = sum_ref[...] / jnp.maximum(deg_ref[...], 1.0)
    out = _dotT(mean, wl_ref[...]) + b_ref[...] + _dotT(xdst_ref[...], wr_ref[...])
    out_ref[...] = jnp.maximum(out, 0.0)


def _sage_dense(summed, deg, x_dst, wl, wr, bias, n_dst):
    grid = (n_dst // BD,)
    return pl.pallas_call(
        _sage_dense_body,
        grid=grid,
        in_specs=[
            pl.BlockSpec((BD, H), lambda i: (i, 0)),
            pl.BlockSpec((BD, 1), lambda i: (i, 0)),
            pl.BlockSpec((BD, H), lambda i: (i, 0)),
            pl.BlockSpec((H, H), lambda i: (0, 0)),
            pl.BlockSpec((H, H), lambda i: (0, 0)),
            pl.BlockSpec((1, H), lambda i: (0, 0)),
        ],
        out_specs=pl.BlockSpec((BD, H), lambda i: (i, 0)),
        out_shape=jax.ShapeDtypeStruct((n_dst, H), jnp.float32),
    )(summed, deg, x_dst, wl, wr, bias)


def _content_body(x_ref, w_ref, b_ref, out_ref):
    out_ref[...] = _dotT(x_ref[...], w_ref[...]) + b_ref[...]


def _content_encode(x, w, bias):
    grid = (N_CONTENT // BD,)
    return pl.pallas_call(
        _content_body,
        grid=grid,
        in_specs=[
            pl.BlockSpec((BD, E_DIM), lambda i: (i, 0)),
            pl.BlockSpec((H, E_DIM), lambda i: (0, 0)),
            pl.BlockSpec((1, H), lambda i: (0, 0)),
        ],
        out_specs=pl.BlockSpec((BD, H), lambda i: (i, 0)),
        out_shape=jax.ShapeDtypeStruct((N_CONTENT, H), jnp.float32),
    )(x, w, bias)


def _cls_body(u_ref, c_ref, w1u_ref, w1c_ref, b1_ref, w2_ref, b2_ref, out_ref):
    hidden = (_dotT(u_ref[...], w1u_ref[...]) + _dotT(c_ref[...], w1c_ref[...])
              + b1_ref[...])
    hidden = jnp.maximum(hidden, 0.0)
    z = jnp.sum(hidden * w2_ref[...], axis=1, keepdims=True) + b2_ref[0, 0]
    out_ref[...] = jax.nn.sigmoid(z)


def _classifier(u, c, w1, b1, w2, b2):
    w1u = w1[:, :H]
    w1c = w1[:, H:]
    grid = (N_USER // BD,)
    return pl.pallas_call(
        _cls_body,
        grid=grid,
        in_specs=[
            pl.BlockSpec((BD, H), lambda i: (i, 0)),
            pl.BlockSpec((BD, H), lambda i: (i, 0)),
            pl.BlockSpec((H, H), lambda i: (0, 0)),
            pl.BlockSpec((H, H), lambda i: (0, 0)),
            pl.BlockSpec((1, H), lambda i: (0, 0)),
            pl.BlockSpec((1, H), lambda i: (0, 0)),
            pl.BlockSpec((1, 1), lambda i: (0, 0)),
        ],
        out_specs=pl.BlockSpec((BD, 1), lambda i: (i, 0)),
        out_shape=jax.ShapeDtypeStruct((N_USER, 1), jnp.float32),
    )(u, c, w1u, w1c, b1, w2, b2)


def kernel(x_user, x_content, edge_index, lstm_Wih, lstm_Whh, lstm_bih,
           lstm_bhh, Wc, bc, Wl0, Wr0, b0, Wl1, Wr1, b1, cls_W1, cls_b1,
           cls_W2, cls_b2):
    lstm_b = (lstm_bih + lstm_bhh).reshape(1, 4 * H)
    u = _lstm_last(x_user, lstm_Wih, lstm_Whh, lstm_b)
    c = _content_encode(x_content, Wc, bc.reshape(1, H))

    src, dst = edge_index[0], edge_index[1]
    ones = jnp.ones((N_EDGES,), jnp.float32)
    deg_c = jax.ops.segment_sum(ones, dst, num_segments=N_CONTENT).reshape(-1, 1)
    deg_u = jax.ops.segment_sum(ones, src, num_segments=N_USER).reshape(-1, 1)

    for (Wl, Wr, b) in ((Wl0, Wr0, b0), (Wl1, Wr1, b1)):
        sum_c = jax.ops.segment_sum(u[src], dst, num_segments=N_CONTENT)
        sum_u = jax.ops.segment_sum(c[dst], src, num_segments=N_USER)
        c_new = _sage_dense(sum_c, deg_c, c, Wl, Wr, b.reshape(1, H), N_CONTENT)
        u_new = _sage_dense(sum_u, deg_u, u, Wl, Wr, b.reshape(1, H), N_USER)
        u, c = u_new, c_new

    return _classifier(u, c, cls_W1, cls_b1.reshape(1, H),
                       cls_W2, cls_b2.reshape(1, 1))
